# baseline (speedup 1.0000x reference)
import functools

import jax
import jax.numpy as jnp
from jax import lax
from jax.experimental import pallas as pl
from jax.experimental.pallas import tpu as pltpu
from jax.experimental.pallas import tpu_sc as plsc

_WINDOW = 11
_LANES = 16
_BLK = 128


def _sc_window_ids(words, packed):
    mesh = plsc.VectorSubcoreMesh(
        core_axis_name="c", subcore_axis_name="s", num_cores=1, num_subcores=1
    )

    @functools.partial(
        pl.kernel,
        out_type=jax.ShapeDtypeStruct((_LANES,), jnp.int32),
        mesh=mesh,
        scratch_types=[
            pltpu.VMEM((2 * _LANES,), jnp.int32),
            pltpu.VMEM((_LANES,), jnp.int32),
            pltpu.SemaphoreType.DMA,
        ],
    )
    def _ids(words_hbm, packed_hbm, out_hbm, pk_v, idx_v, sem):
        pltpu.sync_copy(packed_hbm, pk_v)
        pltpu.async_copy(words_hbm.at[pk_v.at[pl.ds(0, _LANES)]], idx_v,
                         sem).wait()
        idx_v[...] = idx_v[...] * pk_v[pl.ds(_LANES, _LANES)]
        pltpu.sync_copy(idx_v, out_hbm)

    return _ids(words, packed)


def _tc_extract(table_t, wids):
    embed_dim = table_t.shape[0]
    width = _WINDOW * _BLK

    def _body(wids_ref, wids_vec_ref, tbl_ref, out_ref, blks_ref, sem):
        copies = []
        for j in range(_WINDOW):
            g = (wids_ref[j] // _BLK) * _BLK
            copies.append(pltpu.make_async_copy(
                tbl_ref.at[:, pl.ds(g, _BLK)],
                blks_ref.at[:, pl.ds(j * _BLK, _BLK)], sem))
        for c in copies:
            c.start()
        for c in copies:
            c.wait()
        v = wids_vec_ref[...]
        target = lax.broadcasted_iota(jnp.int32, (_LANES, 1), 0) * _BLK \
            + (v % _BLK).reshape(_LANES, 1)
        onehot = (target == lax.broadcasted_iota(jnp.int32, (_LANES, width), 1)
                  ).astype(jnp.float32)
        res = lax.dot_general(
            onehot, blks_ref[...], (((1,), (1,)), ((), ())),
            precision=lax.Precision.HIGHEST,
            preferred_element_type=jnp.float32)
        out_ref[...] = res[:_WINDOW, :]

    grid_spec = pltpu.PrefetchScalarGridSpec(
        num_scalar_prefetch=1,
        grid=(1,),
        in_specs=[
            pl.BlockSpec((1, _LANES), lambda i, wids_ref: (0, 0)),
            pl.BlockSpec(memory_space=pl.ANY),
        ],
        out_specs=pl.BlockSpec((_WINDOW, embed_dim), lambda i, wids_ref: (0, 0)),
        scratch_shapes=[
            pltpu.VMEM((embed_dim, width), jnp.float32),
            pltpu.SemaphoreType.DMA,
        ],
    )
    return pl.pallas_call(
        _body,
        grid_spec=grid_spec,
        out_shape=jax.ShapeDtypeStruct((_WINDOW, embed_dim), jnp.float32),
    )(wids, wids.reshape(1, _LANES), table_t)


def kernel(table, words, wid, wsize):
    seq_len = words.shape[0]
    table_t = table.T
    pos = (
        jnp.asarray(wid, jnp.int32)
        - jnp.asarray(wsize, jnp.int32)
        + jnp.arange(_LANES, dtype=jnp.int32)
    )
    valid = (pos >= 0) & (pos < seq_len) & (jnp.arange(_LANES) < _WINDOW)
    packed = jnp.concatenate(
        [jnp.clip(pos, 0, seq_len - 1), valid.astype(jnp.int32)])
    wids = _sc_window_ids(words, packed)
    return _tc_extract(table_t, wids)


# speedup vs baseline: 1.0011x; 1.0011x over previous
import functools

import jax
import jax.numpy as jnp
from jax import lax
from jax.experimental import pallas as pl
from jax.experimental.pallas import tpu as pltpu
from jax.experimental.pallas import tpu_sc as plsc

_WINDOW = 11
_LANES = 16
_BLK = 128


def _sc_window_ids(words, pos, seq_len):
    mesh = plsc.VectorSubcoreMesh(
        core_axis_name="c", subcore_axis_name="s", num_cores=1, num_subcores=1
    )

    @functools.partial(
        pl.kernel,
        out_type=jax.ShapeDtypeStruct((_LANES,), jnp.int32),
        mesh=mesh,
        scratch_types=[
            pltpu.VMEM((_LANES,), jnp.int32),
            pltpu.VMEM((_LANES,), jnp.int32),
            pltpu.SemaphoreType.DMA,
        ],
    )
    def _ids(words_hbm, pos_hbm, out_hbm, pos_v, idx_v, sem):
        pltpu.sync_copy(pos_hbm, pos_v)
        p = pos_v[...]
        pos_v[...] = jnp.clip(p, 0, seq_len - 1)
        pltpu.async_copy(words_hbm.at[pos_v], idx_v, sem).wait()
        lane = lax.iota(jnp.int32, _LANES)
        valid = (p >= 0) & (p < seq_len) & (lane < _WINDOW)
        idx_v[...] = jnp.where(valid, idx_v[...], 0)
        pltpu.sync_copy(idx_v, out_hbm)

    return _ids(words, pos)


def _tc_extract(table_t, wids):
    embed_dim = table_t.shape[0]
    width = _WINDOW * _BLK

    def _body(wids_ref, wids_vec_ref, tbl_ref, out_ref, blks_ref, sem):
        copies = []
        for j in range(_WINDOW):
            g = (wids_ref[j] // _BLK) * _BLK
            copies.append(pltpu.make_async_copy(
                tbl_ref.at[:, pl.ds(g, _BLK)],
                blks_ref.at[:, pl.ds(j * _BLK, _BLK)], sem))
        for c in copies:
            c.start()
        for c in copies:
            c.wait()
        v = wids_vec_ref[...]
        target = lax.broadcasted_iota(jnp.int32, (_LANES, 1), 0) * _BLK \
            + (v % _BLK).reshape(_LANES, 1)
        onehot = (target == lax.broadcasted_iota(jnp.int32, (_LANES, width), 1)
                  ).astype(jnp.float32)
        res = lax.dot_general(
            onehot, blks_ref[...], (((1,), (1,)), ((), ())),
            precision=lax.Precision.HIGHEST,
            preferred_element_type=jnp.float32)
        out_ref[...] = res[:_WINDOW, :]

    grid_spec = pltpu.PrefetchScalarGridSpec(
        num_scalar_prefetch=1,
        grid=(1,),
        in_specs=[
            pl.BlockSpec((1, _LANES), lambda i, wids_ref: (0, 0)),
            pl.BlockSpec(memory_space=pl.ANY),
        ],
        out_specs=pl.BlockSpec((_WINDOW, embed_dim), lambda i, wids_ref: (0, 0)),
        scratch_shapes=[
            pltpu.VMEM((embed_dim, width), jnp.float32),
            pltpu.SemaphoreType.DMA,
        ],
    )
    return pl.pallas_call(
        _body,
        grid_spec=grid_spec,
        out_shape=jax.ShapeDtypeStruct((_WINDOW, embed_dim), jnp.float32),
    )(wids, wids.reshape(1, _LANES), table_t)


def kernel(table, words, wid, wsize):
    seq_len = words.shape[0]
    table_t = table.T
    pos = (
        jnp.asarray(wid, jnp.int32)
        - jnp.asarray(wsize, jnp.int32)
        + jnp.arange(_LANES, dtype=jnp.int32)
    )
    wids = _sc_window_ids(words, pos, seq_len)
    return _tc_extract(table_t, wids)
